# baseline (speedup 1.0000x reference)
import functools

import jax
import jax.numpy as jnp
from jax import lax
from jax.experimental import pallas as pl
from jax.experimental.pallas import tpu as pltpu
from jax.experimental.pallas import tpu_sc as plsc

B_, T_, K_ = 16, 1024, 256
M = B_ * T_
NE = 8192
BM = 1024
LANES = 256

_REGIONS = ((0, 2816), (2816, 5632), (5632, 8192))


def _argmax_body(x_ref, e_ref, o_ref):
    x = x_ref[...]
    lane_iota = lax.broadcasted_iota(jnp.int32, (BM, LANES), 1)

    acc_v = None
    acc_i = None
    for lo, hi in _REGIONS:
        w = hi - lo
        e = e_ref[lo:hi, :]
        sim = lax.dot_general(
            x, e, (((1,), (1,)), ((), ())),
            preferred_element_type=jnp.float32)
        rv = sim[:, 0:LANES]
        ri = lane_iota + lo
        for k in range(1, w // LANES):
            b = sim[:, k * LANES:(k + 1) * LANES]
            upd = b > rv
            rv = jnp.maximum(rv, b)
            ri = jnp.where(upd, lane_iota + (lo + k * LANES), ri)
        lm = jnp.max(rv, axis=1, keepdims=True)
        la = jnp.min(jnp.where(rv == lm, ri, NE), axis=1, keepdims=True)
        if acc_v is None:
            acc_v, acc_i = lm, la
        else:
            carry = acc_v.astype(jnp.bfloat16).astype(jnp.float32)
            upd = lm > carry
            acc_v = jnp.where(upd, lm, acc_v)
            acc_i = jnp.where(upd, la, acc_i)
    o_ref[0, 0, :] = acc_i[:, 0]


def _matmul_argmax(x, embd):
    m = x.shape[0]
    return pl.pallas_call(
        _argmax_body,
        grid=(m // BM,),
        in_specs=[
            pl.BlockSpec((BM, K_), lambda i: (i, 0)),
            pl.BlockSpec((NE, K_), lambda i: (0, 0)),
        ],
        out_specs=pl.BlockSpec((1, 1, BM), lambda i: (i, 0, 0)),
        out_shape=jax.ShapeDtypeStruct((m // BM, 1, BM), jnp.int32),
    )(x, embd)


def _make_sc_gather(m):
    info = plsc.get_sparse_core_info()
    nc, ns = info.num_cores, info.num_subcores
    nw = nc * ns
    b_per_w = m // nw
    ch = min(128, b_per_w)
    n_ch = b_per_w // ch
    mesh = plsc.VectorSubcoreMesh(core_axis_name="c", subcore_axis_name="s")

    @functools.partial(
        pl.kernel, mesh=mesh,
        out_type=jax.ShapeDtypeStruct((m, K_), jnp.float32),
        scratch_types=[
            pltpu.VMEM((b_per_w,), jnp.int32),
            pltpu.VMEM((2, ch, K_), jnp.float32),
            pltpu.SemaphoreType.DMA,
            pltpu.SemaphoreType.DMA,
        ],
    )
    def gather_k(embd_hbm, idx_hbm, out_hbm, idx_v, rows_v, sem0, sem1):
        wid = lax.axis_index("s") * nc + lax.axis_index("c")
        base = wid * b_per_w
        pltpu.sync_copy(idx_hbm.at[pl.ds(base, b_per_w)], idx_v)
        sems = (sem0, sem1)
        cps = []
        for c in range(n_ch):
            cps.append(pltpu.async_copy(
                embd_hbm.at[idx_v.at[pl.ds(c * ch, ch)]],
                rows_v.at[c % 2], sems[c % 2]))
            if c > 0:
                cps[c - 1].wait()
                pltpu.sync_copy(rows_v.at[(c - 1) % 2],
                                out_hbm.at[pl.ds(base + (c - 1) * ch, ch)])
        cps[n_ch - 1].wait()
        pltpu.sync_copy(rows_v.at[(n_ch - 1) % 2],
                        out_hbm.at[pl.ds(base + (n_ch - 1) * ch, ch)])

    return gather_k


_sc_gather = _make_sc_gather(M)


def kernel(input, embd):
    x = input.reshape(M, K_)
    idx_flat = _matmul_argmax(x, embd).reshape(M)
    quant = _sc_gather(embd, idx_flat)
    return quant.reshape(B_, T_, K_), idx_flat.reshape(B_, T_)


# speedup vs baseline: 1.0468x; 1.0468x over previous
import functools

import jax
import jax.numpy as jnp
from jax import lax
from jax.experimental import pallas as pl
from jax.experimental.pallas import tpu as pltpu
from jax.experimental.pallas import tpu_sc as plsc

B_, T_, K_ = 16, 1024, 256
M = B_ * T_
NE = 8192
BM = 1024
LANES = 128

_REGIONS = ((0, 2816), (2816, 5632), (5632, 8192))


def _argmax_body(x_ref, e_ref, o_ref):
    x = x_ref[...]
    lane_iota = lax.broadcasted_iota(jnp.int32, (BM, LANES), 1)

    acc_v = None
    acc_i = None
    for lo, hi in _REGIONS:
        w = hi - lo
        e = e_ref[lo:hi, :]
        sim = lax.dot_general(
            x, e, (((1,), (1,)), ((), ())),
            preferred_element_type=jnp.float32)
        rv = sim[:, 0:LANES]
        ri = lane_iota + lo
        for k in range(1, w // LANES):
            b = sim[:, k * LANES:(k + 1) * LANES]
            upd = b > rv
            rv = jnp.maximum(rv, b)
            ri = jnp.where(upd, lane_iota + (lo + k * LANES), ri)
        lm = jnp.max(rv, axis=1, keepdims=True)
        la = jnp.min(jnp.where(rv == lm, ri, NE), axis=1, keepdims=True)
        if acc_v is None:
            acc_v, acc_i = lm, la
        else:
            carry = acc_v.astype(jnp.bfloat16).astype(jnp.float32)
            upd = lm > carry
            acc_v = jnp.where(upd, lm, acc_v)
            acc_i = jnp.where(upd, la, acc_i)
    o_ref[0, 0, :] = acc_i[:, 0]


def _matmul_argmax(x, embd):
    m = x.shape[0]
    return pl.pallas_call(
        _argmax_body,
        grid=(m // BM,),
        in_specs=[
            pl.BlockSpec((BM, K_), lambda i: (i, 0)),
            pl.BlockSpec((NE, K_), lambda i: (0, 0)),
        ],
        out_specs=pl.BlockSpec((1, 1, BM), lambda i: (i, 0, 0)),
        out_shape=jax.ShapeDtypeStruct((m // BM, 1, BM), jnp.int32),
    )(x, embd)


def _make_sc_gather(m):
    info = plsc.get_sparse_core_info()
    nc, ns = info.num_cores, info.num_subcores
    nw = nc * ns
    b_per_w = m // nw
    ch = min(128, b_per_w)
    n_ch = b_per_w // ch
    mesh = plsc.VectorSubcoreMesh(core_axis_name="c", subcore_axis_name="s")

    @functools.partial(
        pl.kernel, mesh=mesh,
        out_type=jax.ShapeDtypeStruct((m, K_), jnp.float32),
        scratch_types=[
            pltpu.VMEM((b_per_w,), jnp.int32),
            pltpu.VMEM((2, ch, K_), jnp.float32),
            pltpu.SemaphoreType.DMA,
            pltpu.SemaphoreType.DMA,
        ],
    )
    def gather_k(embd_hbm, idx_hbm, out_hbm, idx_v, rows_v, sem0, sem1):
        wid = lax.axis_index("s") * nc + lax.axis_index("c")
        base = wid * b_per_w
        pltpu.sync_copy(idx_hbm.at[pl.ds(base, b_per_w)], idx_v)
        sems = (sem0, sem1)
        cps = []
        for c in range(n_ch):
            cps.append(pltpu.async_copy(
                embd_hbm.at[idx_v.at[pl.ds(c * ch, ch)]],
                rows_v.at[c % 2], sems[c % 2]))
            if c > 0:
                cps[c - 1].wait()
                pltpu.sync_copy(rows_v.at[(c - 1) % 2],
                                out_hbm.at[pl.ds(base + (c - 1) * ch, ch)])
        cps[n_ch - 1].wait()
        pltpu.sync_copy(rows_v.at[(n_ch - 1) % 2],
                        out_hbm.at[pl.ds(base + (n_ch - 1) * ch, ch)])

    return gather_k


_sc_gather = _make_sc_gather(M)


def kernel(input, embd):
    x = input.reshape(M, K_)
    idx_flat = _matmul_argmax(x, embd).reshape(M)
    quant = _sc_gather(embd, idx_flat)
    return quant.reshape(B_, T_, K_), idx_flat.reshape(B_, T_)
